# baseline (device time: 59984 ns/iter reference)
import jax
import jax.numpy as jnp
from jax import lax
from jax.experimental import pallas as pl
from jax.experimental.pallas import tpu as pltpu

N_DEV = 4
SUB = 2


def kernel(x, w_mat):
    m, _ = x.shape
    _, n = w_mat.shape
    m_per = m // N_DEV
    n_half = n // 2
    m_sub = m_per // SUB

    scales = [4.0 * 0.5 * float(h + 1) ** 0.5 / 127.0
              for h in range(N_DEV - 1)]

    def body(x_ref, w_ref, out_ref, wb_ref, p_ref, comm_a, comm_b,
             send_a, recv_a, send_b, recv_b):
        my = lax.axis_index("i")
        left = lax.rem(my + N_DEV - 1, N_DEV)
        right = lax.rem(my + 1, N_DEV)

        barrier_sem = pltpu.get_barrier_semaphore()
        pl.semaphore_signal(barrier_sem, inc=1, device_id=(left,),
                            device_id_type=pl.DeviceIdType.MESH)
        pl.semaphore_signal(barrier_sem, inc=1, device_id=(right,),
                            device_id_type=pl.DeviceIdType.MESH)
        wb_ref[:, :] = w_ref[:, :].astype(jnp.bfloat16)
        pl.semaphore_wait(barrier_sem, 2)

        def part(c, s, half):
            rows = pl.ds(c * m_per + s * m_sub, m_sub)
            cols = slice(0, n_half) if half == 0 else slice(n_half, n)
            return jnp.dot(x_ref[rows, :].astype(jnp.bfloat16),
                           wb_ref[:, cols],
                           preferred_element_type=jnp.float32)

        def quant(v, h):
            return jnp.clip(jnp.round(v * (1.0 / scales[h])),
                            -127.0, 127.0).astype(jnp.int8)

        def mk(direction, h, s):
            comm = comm_a if direction == 0 else comm_b
            ssem = send_a if direction == 0 else send_b
            rsem = recv_a if direction == 0 else recv_b
            tgt = right if direction == 0 else left
            return pltpu.make_async_remote_copy(
                src_ref=comm.at[h, s],
                dst_ref=comm.at[h + 1, s],
                send_sem=ssem.at[h, s],
                recv_sem=rsem.at[h, s],
                device_id=(tgt,),
                device_id_type=pl.DeviceIdType.MESH,
            )

        desc = {}
        started = []

        for s in range(SUB):
            for direction, c0 in ((0, left), (1, right)):
                comm = comm_a if direction == 0 else comm_b
                comm[0, s, :, :] = quant(part(c0, s, direction), 0)
                r = mk(direction, 0, s)
                r.start()
                desc[(direction, 0, s)] = r
                started.append(r)

        for h in range(N_DEV - 1):
            c_a = lax.rem(my + 2 * N_DEV - 2 - h, N_DEV)
            c_b = lax.rem(my + 2 + h, N_DEV)
            for s in range(SUB):
                p_ref[0, s, :, :] = part(c_a, s, 0).astype(jnp.bfloat16)
                p_ref[1, s, :, :] = part(c_b, s, 1).astype(jnp.bfloat16)
                for direction in (0, 1):
                    comm = comm_a if direction == 0 else comm_b
                    desc[(direction, h, s)].wait_recv()
                    acc = (comm[h + 1, s, :, :].astype(jnp.float32)
                           * scales[h]
                           + p_ref[direction, s, :, :].astype(jnp.float32))
                    if h < N_DEV - 2:
                        comm[h + 1, s, :, :] = quant(acc, h + 1)
                        r = mk(direction, h + 1, s)
                        r.start()
                        desc[(direction, h + 1, s)] = r
                        started.append(r)
                    else:
                        cols = (slice(0, n_half) if direction == 0
                                else slice(n_half, n))
                        out_ref[pl.ds(s * m_sub, m_sub), cols] = (
                            jnp.maximum(acc, 0.0))

        for r in started:
            r.wait_send()

    return pl.pallas_call(
        body,
        out_shape=jax.ShapeDtypeStruct((m_per, n), jnp.float32),
        in_specs=[
            pl.BlockSpec(memory_space=pltpu.VMEM),
            pl.BlockSpec(memory_space=pltpu.VMEM),
        ],
        out_specs=pl.BlockSpec(memory_space=pltpu.VMEM),
        scratch_shapes=[
            pltpu.VMEM((w_mat.shape[0], n), jnp.bfloat16),
            pltpu.VMEM((2, SUB, m_sub, n_half), jnp.bfloat16),
            pltpu.VMEM((N_DEV, SUB, m_sub, n_half), jnp.int8),
            pltpu.VMEM((N_DEV, SUB, m_sub, n_half), jnp.int8),
            pltpu.SemaphoreType.DMA((N_DEV - 1, SUB)),
            pltpu.SemaphoreType.DMA((N_DEV - 1, SUB)),
            pltpu.SemaphoreType.DMA((N_DEV - 1, SUB)),
            pltpu.SemaphoreType.DMA((N_DEV - 1, SUB)),
        ],
        compiler_params=pltpu.CompilerParams(
            collective_id=0,
            vmem_limit_bytes=100 * 1024 * 1024,
        ),
    )(x, w_mat)


# device time: 59463 ns/iter; 1.0088x vs baseline; 1.0088x over previous
import jax
import jax.numpy as jnp
from jax import lax
from jax.experimental import pallas as pl
from jax.experimental.pallas import tpu as pltpu

N_DEV = 4
SUB = 8


def kernel(x, w_mat):
    m, _ = x.shape
    _, n = w_mat.shape
    m_per = m // N_DEV
    n_half = n // 2
    m_sub = m_per // SUB

    scales = [4.0 * 0.5 * float(h + 1) ** 0.5 / 127.0
              for h in range(N_DEV - 1)]

    def body(x_ref, w_ref, out_ref, wb_ref, p_ref, comm_a, comm_b,
             send_a, recv_a, send_b, recv_b):
        my = lax.axis_index("i")
        left = lax.rem(my + N_DEV - 1, N_DEV)
        right = lax.rem(my + 1, N_DEV)

        barrier_sem = pltpu.get_barrier_semaphore()
        pl.semaphore_signal(barrier_sem, inc=1, device_id=(left,),
                            device_id_type=pl.DeviceIdType.MESH)
        pl.semaphore_signal(barrier_sem, inc=1, device_id=(right,),
                            device_id_type=pl.DeviceIdType.MESH)
        wb_ref[:, :] = w_ref[:, :].astype(jnp.bfloat16)
        pl.semaphore_wait(barrier_sem, 2)

        def part(c, s, half):
            rows = pl.ds(c * m_per + s * m_sub, m_sub)
            cols = slice(0, n_half) if half == 0 else slice(n_half, n)
            return jnp.dot(x_ref[rows, :].astype(jnp.bfloat16),
                           wb_ref[:, cols],
                           preferred_element_type=jnp.float32)

        def quant(v, h):
            return jnp.clip(jnp.round(v * (1.0 / scales[h])),
                            -127.0, 127.0).astype(jnp.int8)

        def mk(direction, h, s):
            comm = comm_a if direction == 0 else comm_b
            ssem = send_a if direction == 0 else send_b
            rsem = recv_a if direction == 0 else recv_b
            tgt = right if direction == 0 else left
            return pltpu.make_async_remote_copy(
                src_ref=comm.at[h, s],
                dst_ref=comm.at[h + 1, s],
                send_sem=ssem.at[h, s],
                recv_sem=rsem.at[h, s],
                device_id=(tgt,),
                device_id_type=pl.DeviceIdType.MESH,
            )

        desc = {}
        started = []

        for s in range(SUB):
            for direction, c0 in ((0, left), (1, right)):
                comm = comm_a if direction == 0 else comm_b
                comm[0, s, :, :] = quant(part(c0, s, direction), 0)
                r = mk(direction, 0, s)
                r.start()
                desc[(direction, 0, s)] = r
                started.append(r)

        for h in range(N_DEV - 1):
            c_a = lax.rem(my + 2 * N_DEV - 2 - h, N_DEV)
            c_b = lax.rem(my + 2 + h, N_DEV)
            for s in range(SUB):
                p_ref[0, s, :, :] = part(c_a, s, 0).astype(jnp.bfloat16)
                p_ref[1, s, :, :] = part(c_b, s, 1).astype(jnp.bfloat16)
                for direction in (0, 1):
                    comm = comm_a if direction == 0 else comm_b
                    desc[(direction, h, s)].wait_recv()
                    acc = (comm[h + 1, s, :, :].astype(jnp.float32)
                           * scales[h]
                           + p_ref[direction, s, :, :].astype(jnp.float32))
                    if h < N_DEV - 2:
                        comm[h + 1, s, :, :] = quant(acc, h + 1)
                        r = mk(direction, h + 1, s)
                        r.start()
                        desc[(direction, h + 1, s)] = r
                        started.append(r)
                    else:
                        cols = (slice(0, n_half) if direction == 0
                                else slice(n_half, n))
                        out_ref[pl.ds(s * m_sub, m_sub), cols] = (
                            jnp.maximum(acc, 0.0))

        for r in started:
            r.wait_send()

    return pl.pallas_call(
        body,
        out_shape=jax.ShapeDtypeStruct((m_per, n), jnp.float32),
        in_specs=[
            pl.BlockSpec(memory_space=pltpu.VMEM),
            pl.BlockSpec(memory_space=pltpu.VMEM),
        ],
        out_specs=pl.BlockSpec(memory_space=pltpu.VMEM),
        scratch_shapes=[
            pltpu.VMEM((w_mat.shape[0], n), jnp.bfloat16),
            pltpu.VMEM((2, SUB, m_sub, n_half), jnp.bfloat16),
            pltpu.VMEM((N_DEV, SUB, m_sub, n_half), jnp.int8),
            pltpu.VMEM((N_DEV, SUB, m_sub, n_half), jnp.int8),
            pltpu.SemaphoreType.DMA((N_DEV - 1, SUB)),
            pltpu.SemaphoreType.DMA((N_DEV - 1, SUB)),
            pltpu.SemaphoreType.DMA((N_DEV - 1, SUB)),
            pltpu.SemaphoreType.DMA((N_DEV - 1, SUB)),
        ],
        compiler_params=pltpu.CompilerParams(
            collective_id=0,
            vmem_limit_bytes=100 * 1024 * 1024,
        ),
    )(x, w_mat)


# device time: 58696 ns/iter; 1.0219x vs baseline; 1.0131x over previous
import jax
import jax.numpy as jnp
from jax import lax
from jax.experimental import pallas as pl
from jax.experimental.pallas import tpu as pltpu

N_DEV = 4
SUB = 4


def kernel(x, w_mat):
    m, _ = x.shape
    _, n = w_mat.shape
    m_per = m // N_DEV
    n_half = n // 2
    m_sub = m_per // SUB

    scales = [4.0 * 0.5 * float(h + 1) ** 0.5 / 127.0
              for h in range(N_DEV - 1)]

    def body(x_ref, w_ref, out_ref, wb_ref, p_ref, comm_a, comm_b,
             send_a, recv_a, send_b, recv_b):
        my = lax.axis_index("i")
        left = lax.rem(my + N_DEV - 1, N_DEV)
        right = lax.rem(my + 1, N_DEV)

        barrier_sem = pltpu.get_barrier_semaphore()
        pl.semaphore_signal(barrier_sem, inc=1, device_id=(left,),
                            device_id_type=pl.DeviceIdType.MESH)
        pl.semaphore_signal(barrier_sem, inc=1, device_id=(right,),
                            device_id_type=pl.DeviceIdType.MESH)
        wb_ref[:, :] = w_ref[:, :].astype(jnp.bfloat16)
        pl.semaphore_wait(barrier_sem, 2)

        def part(c, s, half):
            rows = pl.ds(c * m_per + s * m_sub, m_sub)
            cols = slice(0, n_half) if half == 0 else slice(n_half, n)
            return jnp.dot(x_ref[rows, :].astype(jnp.bfloat16),
                           wb_ref[:, cols],
                           preferred_element_type=jnp.float32)

        def quant(v, h):
            return jnp.clip(jnp.round(v * (1.0 / scales[h])),
                            -127.0, 127.0).astype(jnp.int8)

        def mk(direction, h, s):
            comm = comm_a if direction == 0 else comm_b
            ssem = send_a if direction == 0 else send_b
            rsem = recv_a if direction == 0 else recv_b
            tgt = right if direction == 0 else left
            return pltpu.make_async_remote_copy(
                src_ref=comm.at[h, s],
                dst_ref=comm.at[h + 1, s],
                send_sem=ssem.at[h, s],
                recv_sem=rsem.at[h, s],
                device_id=(tgt,),
                device_id_type=pl.DeviceIdType.MESH,
            )

        desc = {}
        started = []

        for s in range(SUB):
            for direction, c0 in ((0, left), (1, right)):
                comm = comm_a if direction == 0 else comm_b
                comm[0, s, :, :] = quant(part(c0, s, direction), 0)
                r = mk(direction, 0, s)
                r.start()
                desc[(direction, 0, s)] = r
                started.append(r)

        for h in range(N_DEV - 1):
            c_a = lax.rem(my + 2 * N_DEV - 2 - h, N_DEV)
            c_b = lax.rem(my + 2 + h, N_DEV)
            inv_next = 1.0 / scales[h + 1] if h < N_DEV - 2 else 1.0
            r_h = scales[h] * inv_next
            for s in range(SUB):
                p_ref[0, s, :, :] = (part(c_a, s, 0)
                                     * inv_next).astype(jnp.bfloat16)
                p_ref[1, s, :, :] = (part(c_b, s, 1)
                                     * inv_next).astype(jnp.bfloat16)
                for direction in (0, 1):
                    comm = comm_a if direction == 0 else comm_b
                    desc[(direction, h, s)].wait_recv()
                    acc = (comm[h + 1, s, :, :].astype(jnp.float32)
                           * r_h
                           + p_ref[direction, s, :, :].astype(jnp.float32))
                    if h < N_DEV - 2:
                        comm[h + 1, s, :, :] = jnp.clip(
                            jnp.round(acc), -127.0, 127.0).astype(jnp.int8)
                        r = mk(direction, h + 1, s)
                        r.start()
                        desc[(direction, h + 1, s)] = r
                        started.append(r)
                    else:
                        cols = (slice(0, n_half) if direction == 0
                                else slice(n_half, n))
                        out_ref[pl.ds(s * m_sub, m_sub), cols] = (
                            jnp.maximum(acc, 0.0))

        for r in started:
            r.wait_send()

    return pl.pallas_call(
        body,
        out_shape=jax.ShapeDtypeStruct((m_per, n), jnp.float32),
        in_specs=[
            pl.BlockSpec(memory_space=pltpu.VMEM),
            pl.BlockSpec(memory_space=pltpu.VMEM),
        ],
        out_specs=pl.BlockSpec(memory_space=pltpu.VMEM),
        scratch_shapes=[
            pltpu.VMEM((w_mat.shape[0], n), jnp.bfloat16),
            pltpu.VMEM((2, SUB, m_sub, n_half), jnp.bfloat16),
            pltpu.VMEM((N_DEV, SUB, m_sub, n_half), jnp.int8),
            pltpu.VMEM((N_DEV, SUB, m_sub, n_half), jnp.int8),
            pltpu.SemaphoreType.DMA((N_DEV - 1, SUB)),
            pltpu.SemaphoreType.DMA((N_DEV - 1, SUB)),
            pltpu.SemaphoreType.DMA((N_DEV - 1, SUB)),
            pltpu.SemaphoreType.DMA((N_DEV - 1, SUB)),
        ],
        compiler_params=pltpu.CompilerParams(
            collective_id=0,
            vmem_limit_bytes=100 * 1024 * 1024,
        ),
    )(x, w_mat)
